# baseline (speedup 1.0000x reference)
import functools

import jax
import jax.numpy as jnp
from jax import lax
from jax.experimental import pallas as pl
from jax.experimental.pallas import tpu as pltpu
from jax.experimental.pallas import tpu_sc as plsc
from jax.experimental.layout import Format, Layout, with_layout_constraint

VOCAB = 1000000
EMBED = 64
SEQ = 200
BATCH = 1024

NC = 2
NS = 16
L = 16
NW = NC * NS
TOK = BATCH * SEQ
PER_W = TOK // NW
C = 320
NCHUNK = PER_W // C
PAIR_V = VOCAB // 2
WIDE = 2 * EMBED
GATHER_SPLITS = ((0, 128), (128, 128), (256, 64))


def _prep_body(seg_tab_ref, pos_ref, seg_lab_ref, combo_ref, cidx_ref):
    combo = seg_tab_ref[:][:, None, :] + pos_ref[:][None, :, :]
    combo_ref[:] = combo.reshape(5 * SEQ, EMBED)
    pos_ids = lax.broadcasted_iota(jnp.int32, (BATCH, SEQ), 1)
    cidx_ref[:] = seg_lab_ref[:] * SEQ + pos_ids


_prep = pl.pallas_call(
    _prep_body,
    out_shape=(
        jax.ShapeDtypeStruct((5 * SEQ, EMBED), jnp.float32),
        jax.ShapeDtypeStruct((BATCH, SEQ), jnp.int32),
    ),
)


def _sc_body(seq_hbm, cidx_hbm, tok_tab, combo_hbm, out_hbm,
             sidx_v, tpair_v, cidx_v, tok_v, cmb_v, res_v, sem_t, sem_c):
    wid = lax.axis_index("s") * NC + lax.axis_index("c")

    def chunk(i, carry):
        tok0 = wid * PER_W + i * C
        pltpu.sync_copy(seq_hbm.at[pl.ds(tok0, C)], sidx_v)
        pltpu.sync_copy(cidx_hbm.at[pl.ds(tok0, C)], cidx_v)

        def shift_body(v, c2):
            sl = pl.ds(v * L, L)
            tpair_v[sl] = lax.shift_right_logical(sidx_v[sl], 1)
            return c2

        lax.fori_loop(0, C // L, shift_body, 0)

        descs = []
        for off, sz in GATHER_SPLITS:
            sl = pl.ds(off, sz)
            descs.append(pltpu.async_copy(
                tok_tab.at[tpair_v.at[sl]], tok_v.at[sl], sem_t))
            descs.append(pltpu.async_copy(
                combo_hbm.at[cidx_v.at[sl]], cmb_v.at[sl], sem_c))
        for d in descs:
            d.wait()

        def add_group(g, c2):
            offs = lax.bitwise_and(sidx_v[pl.ds(g * L, L)], 1) * EMBED
            for k in range(L):
                r = g * L + k
                off = offs[k]
                for j in range(EMBED // L):
                    res_v[r, pl.ds(j * L, L)] = (
                        tok_v[r, pl.ds(off + j * L, L)]
                        + cmb_v[r, pl.ds(j * L, L)])
            return c2

        lax.fori_loop(0, C // L, add_group, 0)
        pltpu.sync_copy(res_v, out_hbm.at[pl.ds(tok0, C)])
        return carry

    lax.fori_loop(0, NCHUNK, chunk, 0)


_sc_embed = functools.partial(
    pl.kernel,
    out_type=jax.ShapeDtypeStruct((TOK, EMBED), jnp.float32),
    mesh=plsc.VectorSubcoreMesh(core_axis_name="c", subcore_axis_name="s"),
    compiler_params=pltpu.CompilerParams(use_tc_tiling_on_sc=False),
    scratch_types=[
        pltpu.VMEM((C,), jnp.int32),
        pltpu.VMEM((C,), jnp.int32),
        pltpu.VMEM((C,), jnp.int32),
        pltpu.VMEM((C, WIDE), jnp.float32),
        pltpu.VMEM((C, WIDE), jnp.float32),
        pltpu.VMEM((C, EMBED), jnp.float32),
        pltpu.SemaphoreType.DMA,
        pltpu.SemaphoreType.DMA,
    ],
)(_sc_body)

@jax.jit
def kernel(sequence, segment_label, token_table, seg_table, pos_embed):
    row_major_2d = Layout(major_to_minor=(0, 1))
    combo, cidx = _prep(seg_table, pos_embed, segment_label.astype(jnp.int32))
    tok_pair = with_layout_constraint(
        jnp.reshape(token_table, (PAIR_V, WIDE)), row_major_2d)
    combo_wide = with_layout_constraint(
        jnp.concatenate([combo, combo], axis=1), row_major_2d)
    seq1 = sequence.astype(jnp.int32).reshape(TOK)
    cidx1 = cidx.reshape(TOK)
    out = _sc_embed(seq1, cidx1, tok_pair, combo_wide)
    return out.reshape(BATCH, SEQ, EMBED)


# speedup vs baseline: 1.1818x; 1.1818x over previous
import functools

import jax
import jax.numpy as jnp
from jax import lax
from jax.experimental import pallas as pl
from jax.experimental.pallas import tpu as pltpu
from jax.experimental.pallas import tpu_sc as plsc
from jax.experimental.layout import Format, Layout, with_layout_constraint

VOCAB = 1000000
EMBED = 64
SEQ = 200
BATCH = 1024

NC = 2
NS = 16
L = 16
NW = NC * NS
TOK = BATCH * SEQ
ROWS_W = BATCH // NW
RC = 4
C = RC * SEQ
NCHUNK = ROWS_W // RC
GATHER_SPLITS = ((0, 80), (80, 80), (160, 40))


def _prep_body(seg_tab_ref, pos_ref, seg_lab_ref, combo_ref, cidx_ref):
    combo = seg_tab_ref[:][:, None, :] + pos_ref[:][None, :, :]
    combo_ref[:] = combo.reshape(5 * SEQ, EMBED)
    pos_ids = lax.broadcasted_iota(jnp.int32, (BATCH, SEQ), 1)
    cidx_ref[:] = seg_lab_ref[:] * SEQ + pos_ids


_prep = pl.pallas_call(
    _prep_body,
    out_shape=(
        jax.ShapeDtypeStruct((5 * SEQ, EMBED), jnp.float32),
        jax.ShapeDtypeStruct((BATCH, SEQ), jnp.int32),
    ),
)


def _sc_body(seq_hbm, cidx_hbm, tok_tab, combo_hbm, out_hbm,
             tidx_v, cidx_v, tok_v, cmb_v, sem_t, sem_c):
    wid = lax.axis_index("s") * NC + lax.axis_index("c")

    def chunk(i, carry):
        row0 = wid * ROWS_W + i * RC
        tok0 = row0 * SEQ
        pltpu.sync_copy(seq_hbm.at[pl.ds(tok0, C)], tidx_v)
        pltpu.sync_copy(cidx_hbm.at[pl.ds(tok0, C)], cidx_v)
        descs = []
        for b in range(RC):
            for l0, sz in GATHER_SPLITS:
                isl = pl.ds(b * SEQ + l0, sz)
                dsl = pl.ds(l0, sz)
                descs.append(pltpu.async_copy(
                    tok_tab.at[tidx_v.at[isl]], tok_v.at[b, dsl], sem_t))
                descs.append(pltpu.async_copy(
                    combo_hbm.at[cidx_v.at[isl]], cmb_v.at[b, dsl], sem_c))
        for d in descs:
            d.wait()

        def add_row(l, c2):
            for b in range(RC):
                for j in range(EMBED // L):
                    sl = pl.ds(j * L, L)
                    tok_v[b, l, sl] = tok_v[b, l, sl] + cmb_v[b, l, sl]
            return c2

        lax.fori_loop(0, SEQ, add_row, 0)
        pltpu.sync_copy(tok_v, out_hbm.at[pl.ds(row0, RC)])
        return carry

    lax.fori_loop(0, NCHUNK, chunk, 0)


_sc_embed = functools.partial(
    pl.kernel,
    out_type=jax.ShapeDtypeStruct((BATCH, SEQ, EMBED), jnp.float32),
    mesh=plsc.VectorSubcoreMesh(core_axis_name="c", subcore_axis_name="s"),
    compiler_params=pltpu.CompilerParams(use_tc_tiling_on_sc=False),
    scratch_types=[
        pltpu.VMEM((C,), jnp.int32),
        pltpu.VMEM((C,), jnp.int32),
        pltpu.VMEM((RC, SEQ, EMBED), jnp.float32),
        pltpu.VMEM((RC, SEQ, EMBED), jnp.float32),
        pltpu.SemaphoreType.DMA,
        pltpu.SemaphoreType.DMA,
    ],
)(_sc_body)

_LIN_1D = Layout(major_to_minor=(0,))


def kernel_impl(sequence, segment_label, token_table, seg_table, pos_embed):
    combo, cidx = _prep(seg_table, pos_embed, segment_label.astype(jnp.int32))
    lin2d = Layout(major_to_minor=(0, 1), tiling=())
    tok_rm = token_table
    combo_rm = combo
    seq1 = with_layout_constraint(
        jnp.reshape(sequence.astype(jnp.int32), (TOK,)), _LIN_1D)
    cidx1 = with_layout_constraint(jnp.reshape(cidx, (TOK,)), _LIN_1D)
    return _sc_embed(seq1, cidx1, tok_rm, combo_rm)


_jitted = None


def kernel(sequence, segment_label, token_table, seg_table, pos_embed):
    global _jitted
    if _jitted is None:
        sharding = getattr(token_table, "sharding", None)
        if sharding is not None:
            out_fmt = Format(
                Layout(major_to_minor=(0, 1, 2), tiling=((8,), (1024,))),
                sharding)
            _jitted = jax.jit(kernel_impl, out_shardings=out_fmt)
        else:
            _jitted = jax.jit(kernel_impl)
    return _jitted(sequence, segment_label, token_table, seg_table, pos_embed)
